# Initial kernel scaffold; baseline (speedup 1.0000x reference)
import jax
import jax.numpy as jnp
from jax.experimental import pallas as pl


def kernel(x, edge_index, W1, b1, W2, b2):
    raise NotImplementedError("write your pallas kernel here")


# speedup vs baseline: 2.4892x; 2.4892x over previous
import functools

import jax
import jax.numpy as jnp
from jax import lax
from jax.experimental import pallas as pl
from jax.experimental.pallas import tpu as pltpu
from jax.experimental.pallas import tpu_sc as plsc

N = 10000
E = 160000
D_IN = 128
H1 = 64
H2 = 16

NC, NS = 2, 16
NW = NC * NS
EW = 128
R = E // EW
RPT = (R + NW - 1) // NW
SLAB = 624
TAIL = N - SLAB * NS
TOFF = SLAB * NS
DEGW = 16

F32 = jnp.float32


def _mesh():
    return plsc.VectorSubcoreMesh(core_axis_name="c", subcore_axis_name="s")


_SC_PARAMS = pltpu.CompilerParams(use_tc_tiling_on_sc=False)

_GUARD = 32768


def _sc_degrees(src1d, dst1d, ones_rows, zeros_deg):
    @functools.partial(
        pl.kernel,
        out_type=jax.ShapeDtypeStruct((NC, 2, N, DEGW), F32),
        mesh=_mesh(),
        compiler_params=_SC_PARAMS,
        scratch_types=[
            pltpu.VMEM((EW,), jnp.int32),
            pltpu.VMEM((EW,), jnp.int32),
            pltpu.VMEM((EW, DEGW), F32),
            pltpu.VMEM_SHARED((_GUARD,), F32),
            pltpu.VMEM_SHARED((N, DEGW), F32),
            pltpu.VMEM_SHARED((N, DEGW), F32),
        ],
    )
    def k(src_h, dst_h, ones_h, zeros_h, out_h, sidx, didx, ones_v, _g, acc_s, acc_d):
        cid = lax.axis_index("c")
        sid = lax.axis_index("s")
        wid = sid * NC + cid
        sl = pl.ds(sid * SLAB, SLAB)
        tl = pl.ds(TOFF, TAIL)
        pltpu.sync_copy(zeros_h.at[sl], acc_s.at[sl])
        pltpu.sync_copy(zeros_h.at[sl], acc_d.at[sl])

        @pl.when(sid == NS - 1)
        def _ztail():
            pltpu.sync_copy(zeros_h.at[tl], acc_s.at[tl])
            pltpu.sync_copy(zeros_h.at[tl], acc_d.at[tl])

        pltpu.sync_copy(ones_h, ones_v)
        plsc.subcore_barrier()

        def body(kk, carry):
            r = kk * NW + wid

            @pl.when(r < R)
            def _do():
                pltpu.sync_copy(src_h.at[pl.ds(r * EW, EW)], sidx)
                pltpu.sync_copy(dst_h.at[pl.ds(r * EW, EW)], didx)
                pltpu.sync_copy(ones_v, acc_s.at[sidx], add=True)
                pltpu.sync_copy(ones_v, acc_d.at[didx], add=True)

            return carry

        lax.fori_loop(0, RPT, body, None)
        plsc.subcore_barrier()
        pltpu.sync_copy(acc_s.at[sl], out_h.at[cid, 0, sl])
        pltpu.sync_copy(acc_d.at[sl], out_h.at[cid, 1, sl])

        @pl.when(sid == NS - 1)
        def _otail():
            pltpu.sync_copy(acc_s.at[tl], out_h.at[cid, 0, tl])
            pltpu.sync_copy(acc_d.at[tl], out_h.at[cid, 1, tl])

    return k(src1d, dst1d, ones_rows, zeros_deg)


def _sc_aggregate(h, src1d, dst1d, zeros_nd, D):
    @functools.partial(
        pl.kernel,
        out_type=jax.ShapeDtypeStruct((NC, N, D), F32),
        mesh=_mesh(),
        compiler_params=_SC_PARAMS,
        scratch_types=[
            pltpu.VMEM((EW,), jnp.int32),
            pltpu.VMEM((EW,), jnp.int32),
            pltpu.VMEM((EW, D), F32),
            pltpu.VMEM_SHARED((_GUARD,), F32),
            pltpu.VMEM_SHARED((N, D), F32),
        ],
    )
    def k(h_h, src_h, dst_h, zeros_h, out_h, sidx, didx, rows, _g, acc):
        cid = lax.axis_index("c")
        sid = lax.axis_index("s")
        wid = sid * NC + cid
        sl = pl.ds(sid * SLAB, SLAB)
        tl = pl.ds(TOFF, TAIL)
        pltpu.sync_copy(zeros_h.at[sl], acc.at[sl])

        @pl.when(sid == NS - 1)
        def _ztail():
            pltpu.sync_copy(zeros_h.at[tl], acc.at[tl])

        plsc.subcore_barrier()

        def body(kk, carry):
            r = kk * NW + wid

            @pl.when(r < R)
            def _do():
                pltpu.sync_copy(src_h.at[pl.ds(r * EW, EW)], sidx)
                pltpu.sync_copy(h_h.at[sidx], rows)
                pltpu.sync_copy(dst_h.at[pl.ds(r * EW, EW)], didx)
                pltpu.sync_copy(rows, acc.at[didx], add=True)

            return carry

        lax.fori_loop(0, RPT, body, None)
        plsc.subcore_barrier()
        pltpu.sync_copy(acc.at[sl], out_h.at[cid, sl])

        @pl.when(sid == NS - 1)
        def _otail():
            pltpu.sync_copy(acc.at[tl], out_h.at[cid, tl])

    return k(h, src1d, dst1d, zeros_nd)


_BM = 1000


def _tc_layer1(x, W1, hist):
    def body(x_ref, w_ref, hs_ref, o_ref):
        deg = hs_ref[0, 0, :, 0] + hs_ref[1, 0, :, 0]
        norm = lax.rsqrt(jnp.maximum(deg, 1.0))
        o_ref[...] = (
            jnp.dot(x_ref[...], w_ref[...], preferred_element_type=F32,
                    precision=lax.Precision.HIGHEST)
            * norm[:, None]
        )

    return pl.pallas_call(
        body,
        grid=(N // _BM,),
        in_specs=[
            pl.BlockSpec((_BM, D_IN), lambda i: (i, 0)),
            pl.BlockSpec((D_IN, H1), lambda i: (0, 0)),
            pl.BlockSpec((NC, 1, _BM, DEGW), lambda i: (0, 0, i, 0)),
        ],
        out_specs=pl.BlockSpec((_BM, H1), lambda i: (i, 0)),
        out_shape=jax.ShapeDtypeStruct((N, H1), F32),
    )(x, W1, hist)


def _tc_layer2(agg1, hist, b1, W2):
    def body(p_ref, hin_ref, hout_ref, b_ref, w_ref, o_ref):
        s = p_ref[0] + p_ref[1]
        din = hin_ref[0, 0, :, 0] + hin_ref[1, 0, :, 0]
        dout = hout_ref[0, 0, :, 0] + hout_ref[1, 0, :, 0]
        h = s * lax.rsqrt(jnp.maximum(din, 1.0))[:, None] + b_ref[0][None, :]
        h = jnp.maximum(h, 0.0)
        o_ref[...] = (
            jnp.dot(h, w_ref[...], preferred_element_type=F32,
                    precision=lax.Precision.HIGHEST)
            * lax.rsqrt(jnp.maximum(dout, 1.0))[:, None]
        )

    return pl.pallas_call(
        body,
        grid=(N // _BM,),
        in_specs=[
            pl.BlockSpec((NC, _BM, H1), lambda i: (0, i, 0)),
            pl.BlockSpec((NC, 1, _BM, DEGW), lambda i: (0, 1, i, 0)),
            pl.BlockSpec((NC, 1, _BM, DEGW), lambda i: (0, 0, i, 0)),
            pl.BlockSpec((1, H1), lambda i: (0, 0)),
            pl.BlockSpec((H1, H2), lambda i: (0, 0)),
        ],
        out_specs=pl.BlockSpec((_BM, H2), lambda i: (i, 0)),
        out_shape=jax.ShapeDtypeStruct((N, H2), F32),
    )(agg1, hist, hist, b1, W2)


def _tc_z(agg2, hist, b2):
    def body(p_ref, hin_ref, b_ref, o_ref):
        din = hin_ref[0, 0, :, 0] + hin_ref[1, 0, :, 0]
        o_ref[...] = (
            (p_ref[0] + p_ref[1]) * lax.rsqrt(jnp.maximum(din, 1.0))[:, None]
            + b_ref[0][None, :]
        )

    return pl.pallas_call(
        body,
        grid=(N // _BM,),
        in_specs=[
            pl.BlockSpec((NC, _BM, H2), lambda i: (0, i, 0)),
            pl.BlockSpec((NC, 1, _BM, DEGW), lambda i: (0, 1, i, 0)),
            pl.BlockSpec((1, H2), lambda i: (0, 0)),
        ],
        out_specs=pl.BlockSpec((_BM, H2), lambda i: (i, 0)),
        out_shape=jax.ShapeDtypeStruct((N, H2), F32),
    )(agg2, hist, b2)


_DM = 200


def _tc_decoder(z):
    def body(zi_ref, zall_ref, o_ref):
        o_ref[...] = lax.dot_general(
            zi_ref[...], zall_ref[...], (((1,), (1,)), ((), ())),
            preferred_element_type=F32, precision=lax.Precision.HIGHEST)

    return pl.pallas_call(
        body,
        grid=(N // _DM,),
        in_specs=[
            pl.BlockSpec((_DM, H2), lambda i: (i, 0)),
            pl.BlockSpec((N, H2), lambda i: (0, 0)),
        ],
        out_specs=pl.BlockSpec((_DM, N), lambda i: (i, 0)),
        out_shape=jax.ShapeDtypeStruct((N, N), F32),
    )(z, z)


def kernel(x, edge_index, W1, b1, W2, b2):
    src1d = edge_index[0]
    dst1d = edge_index[1]
    zeros_deg = jnp.zeros((N, DEGW), F32)
    zeros_h1 = jnp.zeros((N, H1), F32)
    zeros_h2 = jnp.zeros((N, H2), F32)
    ones_rows = jnp.zeros((EW, DEGW), F32).at[:, 0].set(1.0)

    hist = _sc_degrees(src1d, dst1d, ones_rows, zeros_deg)
    h1p = _tc_layer1(x, W1, hist)
    agg1 = _sc_aggregate(h1p, src1d, dst1d, zeros_h1, H1)
    h2p = _tc_layer2(agg1, hist, b1.reshape(1, H1), W2)
    agg2 = _sc_aggregate(h2p, src1d, dst1d, zeros_h2, H2)
    z = _tc_z(agg2, hist, b2.reshape(1, H2))
    return _tc_decoder(z)


# speedup vs baseline: 3.1221x; 1.2542x over previous
import functools

import jax
import jax.numpy as jnp
from jax import lax
from jax.experimental import pallas as pl
from jax.experimental.pallas import tpu as pltpu
from jax.experimental.pallas import tpu_sc as plsc

N = 10000
E = 160000
D_IN = 128
H1 = 64
H2 = 16

NC, NS = 2, 16
NW = NC * NS
EW = 128
R = E // EW
RPW = R // NW
XT = R - RPW * NW
SLAB = 624
TAIL = N - SLAB * NS
TOFF = SLAB * NS
DEGW = 16

F32 = jnp.float32


def _mesh():
    return plsc.VectorSubcoreMesh(core_axis_name="c", subcore_axis_name="s")


_SC_PARAMS = pltpu.CompilerParams(use_tc_tiling_on_sc=False)

_GUARD = 32768


def _preload_idx(src_h, dst_h, sidx_v, didx_v, wid):
    base = wid * RPW
    pltpu.sync_copy(src_h.at[pl.ds(base, RPW)], sidx_v.at[pl.ds(0, RPW)])
    pltpu.sync_copy(dst_h.at[pl.ds(base, RPW)], didx_v.at[pl.ds(0, RPW)])

    @pl.when(wid < XT)
    def _extra():
        pltpu.sync_copy(src_h.at[pl.ds(RPW * NW + wid, 1)], sidx_v.at[pl.ds(RPW, 1)])
        pltpu.sync_copy(dst_h.at[pl.ds(RPW * NW + wid, 1)], didx_v.at[pl.ds(RPW, 1)])


def _sc_degrees(src2d, dst2d, ones_rows, zeros_deg):
    @functools.partial(
        pl.kernel,
        out_type=jax.ShapeDtypeStruct((NC, 2, N, DEGW), F32),
        mesh=_mesh(),
        compiler_params=_SC_PARAMS,
        scratch_types=[
            pltpu.VMEM((RPW + 1, EW), jnp.int32),
            pltpu.VMEM((RPW + 1, EW), jnp.int32),
            pltpu.VMEM((EW, DEGW), F32),
            pltpu.SemaphoreType.DMA,
            pltpu.SemaphoreType.DMA,
            pltpu.VMEM_SHARED((_GUARD,), F32),
            pltpu.VMEM_SHARED((N, DEGW), F32),
            pltpu.VMEM_SHARED((N, DEGW), F32),
        ],
    )
    def k(src_h, dst_h, ones_h, zeros_h, out_h,
          sidx_v, didx_v, ones_v, sem_s, sem_d, _g, acc_s, acc_d):
        cid = lax.axis_index("c")
        sid = lax.axis_index("s")
        wid = sid * NC + cid
        sl = pl.ds(sid * SLAB, SLAB)
        tl = pl.ds(TOFF, TAIL)
        pltpu.sync_copy(zeros_h.at[sl], acc_s.at[sl])
        pltpu.sync_copy(zeros_h.at[sl], acc_d.at[sl])

        @pl.when(sid == NS - 1)
        def _ztail():
            pltpu.sync_copy(zeros_h.at[tl], acc_s.at[tl])
            pltpu.sync_copy(zeros_h.at[tl], acc_d.at[tl])

        pltpu.sync_copy(ones_h, ones_v)
        _preload_idx(src_h, dst_h, sidx_v, didx_v, wid)
        plsc.subcore_barrier()

        nrows = jnp.where(wid < XT, RPW + 1, RPW)

        def body(j, carry):
            d1 = pltpu.async_copy(ones_v, acc_s.at[sidx_v.at[j]], sem_s, add=True)
            d2 = pltpu.async_copy(ones_v, acc_d.at[didx_v.at[j]], sem_d, add=True)
            d1.wait()
            d2.wait()
            return carry

        lax.fori_loop(0, nrows, body, None)
        plsc.subcore_barrier()
        pltpu.sync_copy(acc_s.at[sl], out_h.at[cid, 0, sl])
        pltpu.sync_copy(acc_d.at[sl], out_h.at[cid, 1, sl])

        @pl.when(sid == NS - 1)
        def _otail():
            pltpu.sync_copy(acc_s.at[tl], out_h.at[cid, 0, tl])
            pltpu.sync_copy(acc_d.at[tl], out_h.at[cid, 1, tl])

    return k(src2d, dst2d, ones_rows, zeros_deg)


def _sc_aggregate(h, src2d, dst2d, zeros_nd, D):
    @functools.partial(
        pl.kernel,
        out_type=jax.ShapeDtypeStruct((NC, N, D), F32),
        mesh=_mesh(),
        compiler_params=_SC_PARAMS,
        scratch_types=[
            pltpu.VMEM((RPW + 1, EW), jnp.int32),
            pltpu.VMEM((RPW + 1, EW), jnp.int32),
            pltpu.VMEM((EW, D), F32),
            pltpu.VMEM((EW, D), F32),
            pltpu.SemaphoreType.DMA,
            pltpu.SemaphoreType.DMA,
            pltpu.SemaphoreType.DMA,
            pltpu.SemaphoreType.DMA,
            pltpu.VMEM_SHARED((_GUARD,), F32),
            pltpu.VMEM_SHARED((N, D), F32),
        ],
    )
    def k(h_h, src_h, dst_h, zeros_h, out_h,
          sidx_v, didx_v, rows_a, rows_b, ga, gb, sa, sb, _g, acc):
        cid = lax.axis_index("c")
        sid = lax.axis_index("s")
        wid = sid * NC + cid
        sl = pl.ds(sid * SLAB, SLAB)
        tl = pl.ds(TOFF, TAIL)
        pltpu.sync_copy(zeros_h.at[sl], acc.at[sl])

        @pl.when(sid == NS - 1)
        def _ztail():
            pltpu.sync_copy(zeros_h.at[tl], acc.at[tl])

        _preload_idx(src_h, dst_h, sidx_v, didx_v, wid)
        plsc.subcore_barrier()

        npairs = jnp.where(wid < XT, (RPW + 1) // 2, RPW // 2)

        def body(j, carry):
            ra = 2 * j
            rb = 2 * j + 1
            dga = pltpu.async_copy(h_h.at[sidx_v.at[ra]], rows_a, ga)
            dgb = pltpu.async_copy(h_h.at[sidx_v.at[rb]], rows_b, gb)
            dga.wait()
            dsa = pltpu.async_copy(rows_a, acc.at[didx_v.at[ra]], sa, add=True)
            dgb.wait()
            dsb = pltpu.async_copy(rows_b, acc.at[didx_v.at[rb]], sb, add=True)
            dsa.wait()
            dsb.wait()
            return carry

        lax.fori_loop(0, npairs, body, None)

        @pl.when(wid >= XT)
        def _odd():
            pltpu.sync_copy(h_h.at[sidx_v.at[RPW - 1]], rows_a)
            pltpu.sync_copy(rows_a, acc.at[didx_v.at[RPW - 1]], add=True)

        plsc.subcore_barrier()
        pltpu.sync_copy(acc.at[sl], out_h.at[cid, sl])

        @pl.when(sid == NS - 1)
        def _otail():
            pltpu.sync_copy(acc.at[tl], out_h.at[cid, tl])

    return k(h, src2d, dst2d, zeros_nd)


_BM = 1000


def _tc_layer1(x, W1, hist):
    def body(x_ref, w_ref, hs_ref, o_ref):
        deg = hs_ref[0, 0, :, 0] + hs_ref[1, 0, :, 0]
        norm = lax.rsqrt(jnp.maximum(deg, 1.0))
        o_ref[...] = (
            jnp.dot(x_ref[...], w_ref[...], preferred_element_type=F32,
                    precision=lax.Precision.HIGHEST)
            * norm[:, None]
        )

    return pl.pallas_call(
        body,
        grid=(N // _BM,),
        in_specs=[
            pl.BlockSpec((_BM, D_IN), lambda i: (i, 0)),
            pl.BlockSpec((D_IN, H1), lambda i: (0, 0)),
            pl.BlockSpec((NC, 1, _BM, DEGW), lambda i: (0, 0, i, 0)),
        ],
        out_specs=pl.BlockSpec((_BM, H1), lambda i: (i, 0)),
        out_shape=jax.ShapeDtypeStruct((N, H1), F32),
    )(x, W1, hist)


def _tc_layer2(agg1, hist, b1, W2):
    def body(p_ref, hin_ref, hout_ref, b_ref, w_ref, o_ref):
        s = p_ref[0] + p_ref[1]
        din = hin_ref[0, 0, :, 0] + hin_ref[1, 0, :, 0]
        dout = hout_ref[0, 0, :, 0] + hout_ref[1, 0, :, 0]
        h = s * lax.rsqrt(jnp.maximum(din, 1.0))[:, None] + b_ref[0][None, :]
        h = jnp.maximum(h, 0.0)
        o_ref[...] = (
            jnp.dot(h, w_ref[...], preferred_element_type=F32,
                    precision=lax.Precision.HIGHEST)
            * lax.rsqrt(jnp.maximum(dout, 1.0))[:, None]
        )

    return pl.pallas_call(
        body,
        grid=(N // _BM,),
        in_specs=[
            pl.BlockSpec((NC, _BM, H1), lambda i: (0, i, 0)),
            pl.BlockSpec((NC, 1, _BM, DEGW), lambda i: (0, 1, i, 0)),
            pl.BlockSpec((NC, 1, _BM, DEGW), lambda i: (0, 0, i, 0)),
            pl.BlockSpec((1, H1), lambda i: (0, 0)),
            pl.BlockSpec((H1, H2), lambda i: (0, 0)),
        ],
        out_specs=pl.BlockSpec((_BM, H2), lambda i: (i, 0)),
        out_shape=jax.ShapeDtypeStruct((N, H2), F32),
    )(agg1, hist, hist, b1, W2)


def _tc_z(agg2, hist, b2):
    def body(p_ref, hin_ref, b_ref, o_ref):
        din = hin_ref[0, 0, :, 0] + hin_ref[1, 0, :, 0]
        o_ref[...] = (
            (p_ref[0] + p_ref[1]) * lax.rsqrt(jnp.maximum(din, 1.0))[:, None]
            + b_ref[0][None, :]
        )

    return pl.pallas_call(
        body,
        grid=(N // _BM,),
        in_specs=[
            pl.BlockSpec((NC, _BM, H2), lambda i: (0, i, 0)),
            pl.BlockSpec((NC, 1, _BM, DEGW), lambda i: (0, 1, i, 0)),
            pl.BlockSpec((1, H2), lambda i: (0, 0)),
        ],
        out_specs=pl.BlockSpec((_BM, H2), lambda i: (i, 0)),
        out_shape=jax.ShapeDtypeStruct((N, H2), F32),
    )(agg2, hist, b2)


_DM = 200


def _tc_decoder(z):
    def body(zi_ref, zall_ref, o_ref):
        o_ref[...] = lax.dot_general(
            zi_ref[...], zall_ref[...], (((1,), (1,)), ((), ())),
            preferred_element_type=F32, precision=lax.Precision.HIGHEST)

    return pl.pallas_call(
        body,
        grid=(N // _DM,),
        in_specs=[
            pl.BlockSpec((_DM, H2), lambda i: (i, 0)),
            pl.BlockSpec((N, H2), lambda i: (0, 0)),
        ],
        out_specs=pl.BlockSpec((_DM, N), lambda i: (i, 0)),
        out_shape=jax.ShapeDtypeStruct((N, N), F32),
    )(z, z)


def kernel(x, edge_index, W1, b1, W2, b2):
    src1d = edge_index[0].reshape(R, EW)
    dst1d = edge_index[1].reshape(R, EW)
    zeros_deg = jnp.zeros((N, DEGW), F32)
    zeros_h1 = jnp.zeros((N, H1), F32)
    zeros_h2 = jnp.zeros((N, H2), F32)
    ones_rows = jnp.zeros((EW, DEGW), F32).at[:, 0].set(1.0)

    hist = _sc_degrees(src1d, dst1d, ones_rows, zeros_deg)
    h1p = _tc_layer1(x, W1, hist)
    agg1 = _sc_aggregate(h1p, src1d, dst1d, zeros_h1, H1)
    h2p = _tc_layer2(agg1, hist, b1.reshape(1, H1), W2)
    agg2 = _sc_aggregate(h2p, src1d, dst1d, zeros_h2, H2)
    z = _tc_z(agg2, hist, b2.reshape(1, H2))
    return _tc_decoder(z)


# speedup vs baseline: 3.1635x; 1.0133x over previous
import functools

import jax
import jax.numpy as jnp
from jax import lax
from jax.experimental import pallas as pl
from jax.experimental.pallas import tpu as pltpu
from jax.experimental.pallas import tpu_sc as plsc

N = 10000
E = 160000
D_IN = 128
H1 = 64
H2 = 16

NC, NS = 2, 16
NW = NC * NS
EW = 128
R = E // EW
RPW = R // NW
XT = R - RPW * NW
SLAB = 624
TAIL = N - SLAB * NS
TOFF = SLAB * NS
DEGW = 16

F32 = jnp.float32


def _mesh():
    return plsc.VectorSubcoreMesh(core_axis_name="c", subcore_axis_name="s")


_SC_PARAMS = pltpu.CompilerParams(use_tc_tiling_on_sc=False)

_GUARD = 32768


def _preload_idx(src_h, dst_h, sidx_v, didx_v, wid):
    base = wid * RPW
    pltpu.sync_copy(src_h.at[pl.ds(base, RPW)], sidx_v.at[pl.ds(0, RPW)])
    pltpu.sync_copy(dst_h.at[pl.ds(base, RPW)], didx_v.at[pl.ds(0, RPW)])

    @pl.when(wid < XT)
    def _extra():
        pltpu.sync_copy(src_h.at[pl.ds(RPW * NW + wid, 1)], sidx_v.at[pl.ds(RPW, 1)])
        pltpu.sync_copy(dst_h.at[pl.ds(RPW * NW + wid, 1)], didx_v.at[pl.ds(RPW, 1)])


def _sc_degrees(src2d, dst2d, ones_rows, zeros_deg):
    @functools.partial(
        pl.kernel,
        out_type=jax.ShapeDtypeStruct((NC, 2, N, DEGW), F32),
        mesh=_mesh(),
        compiler_params=_SC_PARAMS,
        scratch_types=[
            pltpu.VMEM((RPW + 1, EW), jnp.int32),
            pltpu.VMEM((RPW + 1, EW), jnp.int32),
            pltpu.VMEM((EW, DEGW), F32),
            pltpu.SemaphoreType.DMA,
            pltpu.SemaphoreType.DMA,
            pltpu.VMEM_SHARED((_GUARD,), F32),
            pltpu.VMEM_SHARED((N, DEGW), F32),
            pltpu.VMEM_SHARED((N, DEGW), F32),
        ],
    )
    def k(src_h, dst_h, ones_h, zeros_h, out_h,
          sidx_v, didx_v, ones_v, sem_s, sem_d, _g, acc_s, acc_d):
        cid = lax.axis_index("c")
        sid = lax.axis_index("s")
        wid = sid * NC + cid
        sl = pl.ds(sid * SLAB, SLAB)
        tl = pl.ds(TOFF, TAIL)
        pltpu.sync_copy(zeros_h.at[sl], acc_s.at[sl])
        pltpu.sync_copy(zeros_h.at[sl], acc_d.at[sl])

        @pl.when(sid == NS - 1)
        def _ztail():
            pltpu.sync_copy(zeros_h.at[tl], acc_s.at[tl])
            pltpu.sync_copy(zeros_h.at[tl], acc_d.at[tl])

        pltpu.sync_copy(ones_h, ones_v)
        _preload_idx(src_h, dst_h, sidx_v, didx_v, wid)
        plsc.subcore_barrier()

        nrows = jnp.where(wid < XT, RPW + 1, RPW)

        def body(j, carry):
            d1 = pltpu.async_copy(ones_v, acc_s.at[sidx_v.at[j]], sem_s, add=True)
            d2 = pltpu.async_copy(ones_v, acc_d.at[didx_v.at[j]], sem_d, add=True)
            d1.wait()
            d2.wait()
            return carry

        lax.fori_loop(0, nrows, body, None)
        plsc.subcore_barrier()
        pltpu.sync_copy(acc_s.at[sl], out_h.at[cid, 0, sl])
        pltpu.sync_copy(acc_d.at[sl], out_h.at[cid, 1, sl])

        @pl.when(sid == NS - 1)
        def _otail():
            pltpu.sync_copy(acc_s.at[tl], out_h.at[cid, 0, tl])
            pltpu.sync_copy(acc_d.at[tl], out_h.at[cid, 1, tl])

    return k(src2d, dst2d, ones_rows, zeros_deg)


def _sc_aggregate(h, src2d, dst2d, zeros_nd, D):
    @functools.partial(
        pl.kernel,
        out_type=jax.ShapeDtypeStruct((NC, N, D), F32),
        mesh=_mesh(),
        compiler_params=_SC_PARAMS,
        scratch_types=[
            pltpu.VMEM((RPW + 1, EW), jnp.int32),
            pltpu.VMEM((RPW + 1, EW), jnp.int32),
            pltpu.VMEM((EW, D), F32),
            pltpu.VMEM((EW, D), F32),
            pltpu.SemaphoreType.DMA,
            pltpu.SemaphoreType.DMA,
            pltpu.SemaphoreType.DMA,
            pltpu.SemaphoreType.DMA,
            pltpu.VMEM_SHARED((_GUARD,), F32),
            pltpu.VMEM_SHARED((N, D), F32),
        ],
    )
    def k(h_h, src_h, dst_h, zeros_h, out_h,
          sidx_v, didx_v, rows_a, rows_b, ga, gb, sa, sb, _g, acc):
        cid = lax.axis_index("c")
        sid = lax.axis_index("s")
        wid = sid * NC + cid
        sl = pl.ds(sid * SLAB, SLAB)
        tl = pl.ds(TOFF, TAIL)
        pltpu.sync_copy(zeros_h.at[sl], acc.at[sl])

        @pl.when(sid == NS - 1)
        def _ztail():
            pltpu.sync_copy(zeros_h.at[tl], acc.at[tl])

        _preload_idx(src_h, dst_h, sidx_v, didx_v, wid)
        plsc.subcore_barrier()

        npairs = jnp.where(wid < XT, (RPW + 1) // 2, RPW // 2)

        def body(j, carry):
            ra = 2 * j
            rb = 2 * j + 1
            dga = pltpu.async_copy(h_h.at[sidx_v.at[ra]], rows_a, ga)
            dgb = pltpu.async_copy(h_h.at[sidx_v.at[rb]], rows_b, gb)
            dga.wait()
            dsa = pltpu.async_copy(rows_a, acc.at[didx_v.at[ra]], sa, add=True)
            dgb.wait()
            dsb = pltpu.async_copy(rows_b, acc.at[didx_v.at[rb]], sb, add=True)
            dsa.wait()
            dsb.wait()
            return carry

        lax.fori_loop(0, npairs, body, None)

        @pl.when(wid >= XT)
        def _odd():
            pltpu.sync_copy(h_h.at[sidx_v.at[RPW - 1]], rows_a)
            pltpu.sync_copy(rows_a, acc.at[didx_v.at[RPW - 1]], add=True)

        plsc.subcore_barrier()
        pltpu.sync_copy(acc.at[sl], out_h.at[cid, sl])

        @pl.when(sid == NS - 1)
        def _otail():
            pltpu.sync_copy(acc.at[tl], out_h.at[cid, tl])

    return k(h, src2d, dst2d, zeros_nd)


_BM = 1000


def _tc_layer1(x, W1, hist):
    def body(x_ref, w_ref, hs_ref, o_ref):
        deg = hs_ref[0, 0, :, 0] + hs_ref[1, 0, :, 0]
        norm = lax.rsqrt(jnp.maximum(deg, 1.0))
        o_ref[...] = (
            jnp.dot(x_ref[...], w_ref[...], preferred_element_type=F32,
                    precision=lax.Precision.HIGHEST)
            * norm[:, None]
        )

    return pl.pallas_call(
        body,
        grid=(N // _BM,),
        in_specs=[
            pl.BlockSpec((_BM, D_IN), lambda i: (i, 0)),
            pl.BlockSpec((D_IN, H1), lambda i: (0, 0)),
            pl.BlockSpec((NC, 1, _BM, DEGW), lambda i: (0, 0, i, 0)),
        ],
        out_specs=pl.BlockSpec((_BM, H1), lambda i: (i, 0)),
        out_shape=jax.ShapeDtypeStruct((N, H1), F32),
    )(x, W1, hist)


def _tc_layer2(agg1, hist, b1, W2):
    def body(p_ref, hin_ref, hout_ref, b_ref, w_ref, o_ref):
        s = p_ref[0] + p_ref[1]
        din = hin_ref[0, 0, :, 0] + hin_ref[1, 0, :, 0]
        dout = hout_ref[0, 0, :, 0] + hout_ref[1, 0, :, 0]
        h = s * lax.rsqrt(jnp.maximum(din, 1.0))[:, None] + b_ref[0][None, :]
        h = jnp.maximum(h, 0.0)
        o_ref[...] = (
            jnp.dot(h, w_ref[...], preferred_element_type=F32,
                    precision=lax.Precision.HIGHEST)
            * lax.rsqrt(jnp.maximum(dout, 1.0))[:, None]
        )

    return pl.pallas_call(
        body,
        grid=(N // _BM,),
        in_specs=[
            pl.BlockSpec((NC, _BM, H1), lambda i: (0, i, 0)),
            pl.BlockSpec((NC, 1, _BM, DEGW), lambda i: (0, 1, i, 0)),
            pl.BlockSpec((NC, 1, _BM, DEGW), lambda i: (0, 0, i, 0)),
            pl.BlockSpec((1, H1), lambda i: (0, 0)),
            pl.BlockSpec((H1, H2), lambda i: (0, 0)),
        ],
        out_specs=pl.BlockSpec((_BM, H2), lambda i: (i, 0)),
        out_shape=jax.ShapeDtypeStruct((N, H2), F32),
    )(agg1, hist, hist, b1, W2)


def _tc_z(agg2, hist, b2):
    def body(p_ref, hin_ref, b_ref, o_ref):
        din = hin_ref[0, 0, :, 0] + hin_ref[1, 0, :, 0]
        o_ref[...] = (
            (p_ref[0] + p_ref[1]) * lax.rsqrt(jnp.maximum(din, 1.0))[:, None]
            + b_ref[0][None, :]
        )

    return pl.pallas_call(
        body,
        grid=(N // _BM,),
        in_specs=[
            pl.BlockSpec((NC, _BM, H2), lambda i: (0, i, 0)),
            pl.BlockSpec((NC, 1, _BM, DEGW), lambda i: (0, 1, i, 0)),
            pl.BlockSpec((1, H2), lambda i: (0, 0)),
        ],
        out_specs=pl.BlockSpec((_BM, H2), lambda i: (i, 0)),
        out_shape=jax.ShapeDtypeStruct((N, H2), F32),
    )(agg2, hist, b2)


_DM = 400


def _tc_decoder(z):
    def body(zi_ref, zall_ref, o_ref):
        o_ref[...] = lax.dot_general(
            zi_ref[...], zall_ref[...], (((1,), (1,)), ((), ())),
            preferred_element_type=F32, precision=lax.Precision.HIGHEST)

    return pl.pallas_call(
        body,
        grid=(N // _DM,),
        in_specs=[
            pl.BlockSpec((_DM, H2), lambda i: (i, 0)),
            pl.BlockSpec((N, H2), lambda i: (0, 0)),
        ],
        out_specs=pl.BlockSpec((_DM, N), lambda i: (i, 0)),
        out_shape=jax.ShapeDtypeStruct((N, N), F32),
    )(z, z)


def kernel(x, edge_index, W1, b1, W2, b2):
    src1d = edge_index[0].reshape(R, EW)
    dst1d = edge_index[1].reshape(R, EW)
    zeros_deg = jnp.zeros((N, DEGW), F32)
    zeros_h1 = jnp.zeros((N, H1), F32)
    zeros_h2 = jnp.zeros((N, H2), F32)
    ones_rows = jnp.zeros((EW, DEGW), F32).at[:, 0].set(1.0)

    hist = _sc_degrees(src1d, dst1d, ones_rows, zeros_deg)
    h1p = _tc_layer1(x, W1, hist)
    agg1 = _sc_aggregate(h1p, src1d, dst1d, zeros_h1, H1)
    h2p = _tc_layer2(agg1, hist, b1.reshape(1, H1), W2)
    agg2 = _sc_aggregate(h2p, src1d, dst1d, zeros_h2, H2)
    z = _tc_z(agg2, hist, b2.reshape(1, H2))
    return _tc_decoder(z)


# speedup vs baseline: 3.2394x; 1.0240x over previous
import functools

import jax
import jax.numpy as jnp
from jax import lax
from jax.experimental import pallas as pl
from jax.experimental.pallas import tpu as pltpu
from jax.experimental.pallas import tpu_sc as plsc

N = 10000
E = 160000
D_IN = 128
H1 = 64
H2 = 16

NC, NS = 2, 16
NW = NC * NS
EW = 128
R = E // EW
RPW = R // NW
XT = R - RPW * NW
SLAB = 624
TAIL = N - SLAB * NS
TOFF = SLAB * NS
DEGW = 8

F32 = jnp.float32


def _mesh():
    return plsc.VectorSubcoreMesh(core_axis_name="c", subcore_axis_name="s")


_SC_PARAMS = pltpu.CompilerParams(use_tc_tiling_on_sc=False)

_GUARD = 32768


def _preload_idx(src_h, dst_h, sidx_v, didx_v, wid):
    base = wid * RPW
    pltpu.sync_copy(src_h.at[pl.ds(base, RPW)], sidx_v.at[pl.ds(0, RPW)])
    pltpu.sync_copy(dst_h.at[pl.ds(base, RPW)], didx_v.at[pl.ds(0, RPW)])

    @pl.when(wid < XT)
    def _extra():
        pltpu.sync_copy(src_h.at[pl.ds(RPW * NW + wid, 1)], sidx_v.at[pl.ds(RPW, 1)])
        pltpu.sync_copy(dst_h.at[pl.ds(RPW * NW + wid, 1)], didx_v.at[pl.ds(RPW, 1)])


def _sc_degrees(src2d, dst2d, ones_rows, zeros_deg):
    @functools.partial(
        pl.kernel,
        out_type=jax.ShapeDtypeStruct((NC, 2, N, DEGW), F32),
        mesh=_mesh(),
        compiler_params=_SC_PARAMS,
        scratch_types=[
            pltpu.VMEM((RPW + 1, EW), jnp.int32),
            pltpu.VMEM((RPW + 1, EW), jnp.int32),
            pltpu.VMEM((EW, DEGW), F32),
            pltpu.SemaphoreType.DMA,
            pltpu.SemaphoreType.DMA,
            pltpu.VMEM_SHARED((_GUARD,), F32),
            pltpu.VMEM_SHARED((N, DEGW), F32),
            pltpu.VMEM_SHARED((N, DEGW), F32),
        ],
    )
    def k(src_h, dst_h, ones_h, zeros_h, out_h,
          sidx_v, didx_v, ones_v, sem_s, sem_d, _g, acc_s, acc_d):
        cid = lax.axis_index("c")
        sid = lax.axis_index("s")
        wid = sid * NC + cid
        sl = pl.ds(sid * SLAB, SLAB)
        tl = pl.ds(TOFF, TAIL)
        pltpu.sync_copy(zeros_h.at[sl], acc_s.at[sl])
        pltpu.sync_copy(zeros_h.at[sl], acc_d.at[sl])

        @pl.when(sid == NS - 1)
        def _ztail():
            pltpu.sync_copy(zeros_h.at[tl], acc_s.at[tl])
            pltpu.sync_copy(zeros_h.at[tl], acc_d.at[tl])

        pltpu.sync_copy(ones_h, ones_v)
        _preload_idx(src_h, dst_h, sidx_v, didx_v, wid)
        plsc.subcore_barrier()

        nrows = jnp.where(wid < XT, RPW + 1, RPW)

        def body(j, carry):
            d1 = pltpu.async_copy(ones_v, acc_s.at[sidx_v.at[j]], sem_s, add=True)
            d2 = pltpu.async_copy(ones_v, acc_d.at[didx_v.at[j]], sem_d, add=True)
            d1.wait()
            d2.wait()
            return carry

        lax.fori_loop(0, nrows, body, None)
        plsc.subcore_barrier()
        pltpu.sync_copy(acc_s.at[sl], out_h.at[cid, 0, sl])
        pltpu.sync_copy(acc_d.at[sl], out_h.at[cid, 1, sl])

        @pl.when(sid == NS - 1)
        def _otail():
            pltpu.sync_copy(acc_s.at[tl], out_h.at[cid, 0, tl])
            pltpu.sync_copy(acc_d.at[tl], out_h.at[cid, 1, tl])

    return k(src2d, dst2d, ones_rows, zeros_deg)


def _sc_aggregate(h, src2d, dst2d, zeros_nd, D):
    @functools.partial(
        pl.kernel,
        out_type=jax.ShapeDtypeStruct((NC, N, D), F32),
        mesh=_mesh(),
        compiler_params=_SC_PARAMS,
        scratch_types=[
            pltpu.VMEM((RPW + 1, EW), jnp.int32),
            pltpu.VMEM((RPW + 1, EW), jnp.int32),
            pltpu.VMEM((EW, D), F32),
            pltpu.VMEM((EW, D), F32),
            pltpu.VMEM((EW, D), F32),
            pltpu.VMEM((EW, D), F32),
            pltpu.SemaphoreType.DMA,
            pltpu.SemaphoreType.DMA,
            pltpu.SemaphoreType.DMA,
            pltpu.SemaphoreType.DMA,
            pltpu.SemaphoreType.DMA,
            pltpu.SemaphoreType.DMA,
            pltpu.SemaphoreType.DMA,
            pltpu.SemaphoreType.DMA,
            pltpu.VMEM_SHARED((_GUARD,), F32),
            pltpu.VMEM_SHARED((N, D), F32),
        ],
    )
    def k(h_h, src_h, dst_h, zeros_h, out_h,
          sidx_v, didx_v, rows_a, rows_b, rows_c, rows_d,
          ga, gb, gc, gd, sa, sb, sc_, sd, _g, acc):
        cid = lax.axis_index("c")
        sid = lax.axis_index("s")
        wid = sid * NC + cid
        sl = pl.ds(sid * SLAB, SLAB)
        tl = pl.ds(TOFF, TAIL)
        pltpu.sync_copy(zeros_h.at[sl], acc.at[sl])

        @pl.when(sid == NS - 1)
        def _ztail():
            pltpu.sync_copy(zeros_h.at[tl], acc.at[tl])

        _preload_idx(src_h, dst_h, sidx_v, didx_v, wid)
        plsc.subcore_barrier()

        nquads = jnp.where(wid < XT, (RPW + 1) // 4, RPW // 4)
        bufs = (rows_a, rows_b, rows_c, rows_d)
        gsems = (ga, gb, gc, gd)
        ssems = (sa, sb, sc_, sd)

        def body(j, carry):
            gd_ = [pltpu.async_copy(h_h.at[sidx_v.at[4 * j + i]], bufs[i], gsems[i])
                   for i in range(4)]
            sd_ = []
            for i in range(4):
                gd_[i].wait()
                sd_.append(pltpu.async_copy(bufs[i], acc.at[didx_v.at[4 * j + i]],
                                            ssems[i], add=True))
            for i in range(4):
                sd_[i].wait()
            return carry

        lax.fori_loop(0, nquads, body, None)

        @pl.when(wid >= XT)
        def _odd():
            gd_ = [pltpu.async_copy(h_h.at[sidx_v.at[RPW - 3 + i]], bufs[i], gsems[i])
                   for i in range(3)]
            sd_ = []
            for i in range(3):
                gd_[i].wait()
                sd_.append(pltpu.async_copy(bufs[i], acc.at[didx_v.at[RPW - 3 + i]],
                                            ssems[i], add=True))
            for i in range(3):
                sd_[i].wait()

        plsc.subcore_barrier()
        pltpu.sync_copy(acc.at[sl], out_h.at[cid, sl])

        @pl.when(sid == NS - 1)
        def _otail():
            pltpu.sync_copy(acc.at[tl], out_h.at[cid, tl])

    return k(h, src2d, dst2d, zeros_nd)


_BM = 1000


def _tc_layer1(x, W1, hist):
    def body(x_ref, w_ref, hs_ref, o_ref):
        deg = hs_ref[0, 0, :, 0] + hs_ref[1, 0, :, 0]
        norm = lax.rsqrt(jnp.maximum(deg, 1.0))
        o_ref[...] = (
            jnp.dot(x_ref[...], w_ref[...], preferred_element_type=F32,
                    precision=lax.Precision.HIGHEST)
            * norm[:, None]
        )

    return pl.pallas_call(
        body,
        grid=(N // _BM,),
        in_specs=[
            pl.BlockSpec((_BM, D_IN), lambda i: (i, 0)),
            pl.BlockSpec((D_IN, H1), lambda i: (0, 0)),
            pl.BlockSpec((NC, 1, _BM, DEGW), lambda i: (0, 0, i, 0)),
        ],
        out_specs=pl.BlockSpec((_BM, H1), lambda i: (i, 0)),
        out_shape=jax.ShapeDtypeStruct((N, H1), F32),
    )(x, W1, hist)


def _tc_layer2(agg1, hist, b1, W2):
    def body(p_ref, hin_ref, hout_ref, b_ref, w_ref, o_ref):
        s = p_ref[0] + p_ref[1]
        din = hin_ref[0, 0, :, 0] + hin_ref[1, 0, :, 0]
        dout = hout_ref[0, 0, :, 0] + hout_ref[1, 0, :, 0]
        h = s * lax.rsqrt(jnp.maximum(din, 1.0))[:, None] + b_ref[0][None, :]
        h = jnp.maximum(h, 0.0)
        o_ref[...] = (
            jnp.dot(h, w_ref[...], preferred_element_type=F32,
                    precision=lax.Precision.HIGHEST)
            * lax.rsqrt(jnp.maximum(dout, 1.0))[:, None]
        )

    return pl.pallas_call(
        body,
        grid=(N // _BM,),
        in_specs=[
            pl.BlockSpec((NC, _BM, H1), lambda i: (0, i, 0)),
            pl.BlockSpec((NC, 1, _BM, DEGW), lambda i: (0, 1, i, 0)),
            pl.BlockSpec((NC, 1, _BM, DEGW), lambda i: (0, 0, i, 0)),
            pl.BlockSpec((1, H1), lambda i: (0, 0)),
            pl.BlockSpec((H1, H2), lambda i: (0, 0)),
        ],
        out_specs=pl.BlockSpec((_BM, H2), lambda i: (i, 0)),
        out_shape=jax.ShapeDtypeStruct((N, H2), F32),
    )(agg1, hist, hist, b1, W2)


def _tc_z(agg2, hist, b2):
    def body(p_ref, hin_ref, b_ref, o_ref):
        din = hin_ref[0, 0, :, 0] + hin_ref[1, 0, :, 0]
        o_ref[...] = (
            (p_ref[0] + p_ref[1]) * lax.rsqrt(jnp.maximum(din, 1.0))[:, None]
            + b_ref[0][None, :]
        )

    return pl.pallas_call(
        body,
        grid=(N // _BM,),
        in_specs=[
            pl.BlockSpec((NC, _BM, H2), lambda i: (0, i, 0)),
            pl.BlockSpec((NC, 1, _BM, DEGW), lambda i: (0, 1, i, 0)),
            pl.BlockSpec((1, H2), lambda i: (0, 0)),
        ],
        out_specs=pl.BlockSpec((_BM, H2), lambda i: (i, 0)),
        out_shape=jax.ShapeDtypeStruct((N, H2), F32),
    )(agg2, hist, b2)


_DM = 400


def _tc_decoder(z):
    def body(zi_ref, zall_ref, o_ref):
        o_ref[...] = lax.dot_general(
            zi_ref[...], zall_ref[...], (((1,), (1,)), ((), ())),
            preferred_element_type=F32, precision=lax.Precision.HIGHEST)

    return pl.pallas_call(
        body,
        grid=(N // _DM,),
        in_specs=[
            pl.BlockSpec((_DM, H2), lambda i: (i, 0)),
            pl.BlockSpec((N, H2), lambda i: (0, 0)),
        ],
        out_specs=pl.BlockSpec((_DM, N), lambda i: (i, 0)),
        out_shape=jax.ShapeDtypeStruct((N, N), F32),
    )(z, z)


def kernel(x, edge_index, W1, b1, W2, b2):
    src1d = edge_index[0].reshape(R, EW)
    dst1d = edge_index[1].reshape(R, EW)
    zeros_deg = jnp.zeros((N, DEGW), F32)
    zeros_h1 = jnp.zeros((N, H1), F32)
    zeros_h2 = jnp.zeros((N, H2), F32)
    ones_rows = jnp.zeros((EW, DEGW), F32).at[:, 0].set(1.0)

    hist = _sc_degrees(src1d, dst1d, ones_rows, zeros_deg)
    h1p = _tc_layer1(x, W1, hist)
    agg1 = _sc_aggregate(h1p, src1d, dst1d, zeros_h1, H1)
    h2p = _tc_layer2(agg1, hist, b1.reshape(1, H1), W2)
    agg2 = _sc_aggregate(h2p, src1d, dst1d, zeros_h2, H2)
    z = _tc_z(agg2, hist, b2.reshape(1, H2))
    return _tc_decoder(z)


# speedup vs baseline: 3.2533x; 1.0043x over previous
import functools

import jax
import jax.numpy as jnp
from jax import lax
from jax.experimental import pallas as pl
from jax.experimental.pallas import tpu as pltpu
from jax.experimental.pallas import tpu_sc as plsc

N = 10000
E = 160000
D_IN = 128
H1 = 64
H2 = 16

NC, NS = 2, 16
NW = NC * NS
EW = 128
R = E // EW
RPW = R // NW
XT = R - RPW * NW
SLAB = 624
TAIL = N - SLAB * NS
TOFF = SLAB * NS
DEGW = 8

F32 = jnp.float32


def _mesh():
    return plsc.VectorSubcoreMesh(core_axis_name="c", subcore_axis_name="s")


_SC_PARAMS = pltpu.CompilerParams(use_tc_tiling_on_sc=False)

_GUARD = 32768


def _preload_idx(src_h, dst_h, sidx_v, didx_v, wid):
    base = wid * RPW
    pltpu.sync_copy(src_h.at[pl.ds(base, RPW)], sidx_v.at[pl.ds(0, RPW)])
    pltpu.sync_copy(dst_h.at[pl.ds(base, RPW)], didx_v.at[pl.ds(0, RPW)])

    @pl.when(wid < XT)
    def _extra():
        pltpu.sync_copy(src_h.at[pl.ds(RPW * NW + wid, 1)], sidx_v.at[pl.ds(RPW, 1)])
        pltpu.sync_copy(dst_h.at[pl.ds(RPW * NW + wid, 1)], didx_v.at[pl.ds(RPW, 1)])


def _sc_degrees(src2d, dst2d, ones_rows, zeros_deg):
    @functools.partial(
        pl.kernel,
        out_type=jax.ShapeDtypeStruct((NC, 2, N, DEGW), F32),
        mesh=_mesh(),
        compiler_params=_SC_PARAMS,
        scratch_types=[
            pltpu.VMEM((RPW + 1, EW), jnp.int32),
            pltpu.VMEM((RPW + 1, EW), jnp.int32),
            pltpu.VMEM((EW, DEGW), F32),
            pltpu.SemaphoreType.DMA,
            pltpu.SemaphoreType.DMA,
            pltpu.VMEM_SHARED((_GUARD,), F32),
            pltpu.VMEM_SHARED((N, DEGW), F32),
            pltpu.VMEM_SHARED((N, DEGW), F32),
        ],
    )
    def k(src_h, dst_h, ones_h, zeros_h, out_h,
          sidx_v, didx_v, ones_v, sem_s, sem_d, _g, acc_s, acc_d):
        cid = lax.axis_index("c")
        sid = lax.axis_index("s")
        wid = sid * NC + cid
        sl = pl.ds(sid * SLAB, SLAB)
        tl = pl.ds(TOFF, TAIL)
        pltpu.sync_copy(zeros_h.at[sl], acc_s.at[sl])
        pltpu.sync_copy(zeros_h.at[sl], acc_d.at[sl])

        @pl.when(sid == NS - 1)
        def _ztail():
            pltpu.sync_copy(zeros_h.at[tl], acc_s.at[tl])
            pltpu.sync_copy(zeros_h.at[tl], acc_d.at[tl])

        pltpu.sync_copy(ones_h, ones_v)
        _preload_idx(src_h, dst_h, sidx_v, didx_v, wid)
        plsc.subcore_barrier()

        nrows = jnp.where(wid < XT, RPW + 1, RPW)

        def body(j, carry):
            d1 = pltpu.async_copy(ones_v, acc_s.at[sidx_v.at[j]], sem_s, add=True)
            d2 = pltpu.async_copy(ones_v, acc_d.at[didx_v.at[j]], sem_d, add=True)
            d1.wait()
            d2.wait()
            return carry

        lax.fori_loop(0, nrows, body, None)
        plsc.subcore_barrier()
        pltpu.sync_copy(acc_s.at[sl], out_h.at[cid, 0, sl])
        pltpu.sync_copy(acc_d.at[sl], out_h.at[cid, 1, sl])

        @pl.when(sid == NS - 1)
        def _otail():
            pltpu.sync_copy(acc_s.at[tl], out_h.at[cid, 0, tl])
            pltpu.sync_copy(acc_d.at[tl], out_h.at[cid, 1, tl])

    return k(src2d, dst2d, ones_rows, zeros_deg)


def _sc_aggregate(h, src2d, dst2d, zeros_nd, D):
    @functools.partial(
        pl.kernel,
        out_type=jax.ShapeDtypeStruct((NC, N, D), F32),
        mesh=_mesh(),
        compiler_params=_SC_PARAMS,
        scratch_types=[
            pltpu.VMEM((RPW + 1, EW), jnp.int32),
            pltpu.VMEM((RPW + 1, EW), jnp.int32),
            pltpu.VMEM((EW, D), F32),
            pltpu.VMEM((EW, D), F32),
            pltpu.VMEM((EW, D), F32),
            pltpu.VMEM((EW, D), F32),
            pltpu.SemaphoreType.DMA,
            pltpu.SemaphoreType.DMA,
            pltpu.SemaphoreType.DMA,
            pltpu.SemaphoreType.DMA,
            pltpu.SemaphoreType.DMA,
            pltpu.SemaphoreType.DMA,
            pltpu.SemaphoreType.DMA,
            pltpu.SemaphoreType.DMA,
            pltpu.VMEM_SHARED((_GUARD,), F32),
            pltpu.VMEM_SHARED((N, D), F32),
        ],
    )
    def k(h_h, src_h, dst_h, zeros_h, out_h,
          sidx_v, didx_v, rows_a, rows_b, rows_c, rows_d,
          ga, gb, gc, gd, sa, sb, sc_, sd, _g, acc):
        cid = lax.axis_index("c")
        sid = lax.axis_index("s")
        wid = sid * NC + cid
        sl = pl.ds(sid * SLAB, SLAB)
        tl = pl.ds(TOFF, TAIL)
        pltpu.sync_copy(zeros_h.at[sl], acc.at[sl])

        @pl.when(sid == NS - 1)
        def _ztail():
            pltpu.sync_copy(zeros_h.at[tl], acc.at[tl])

        _preload_idx(src_h, dst_h, sidx_v, didx_v, wid)
        plsc.subcore_barrier()

        nquads = jnp.where(wid < XT, (RPW + 1) // 4, RPW // 4)
        bufs = (rows_a, rows_b, rows_c, rows_d)
        gsems = (ga, gb, gc, gd)
        ssems = (sa, sb, sc_, sd)

        def body(j, carry):
            gd_ = [pltpu.async_copy(h_h.at[sidx_v.at[4 * j + i]], bufs[i], gsems[i])
                   for i in range(4)]
            sd_ = []
            for i in range(4):
                gd_[i].wait()
                sd_.append(pltpu.async_copy(bufs[i], acc.at[didx_v.at[4 * j + i]],
                                            ssems[i], add=True))
            for i in range(4):
                sd_[i].wait()
            return carry

        lax.fori_loop(0, nquads, body, None)

        @pl.when(wid >= XT)
        def _odd():
            gd_ = [pltpu.async_copy(h_h.at[sidx_v.at[RPW - 3 + i]], bufs[i], gsems[i])
                   for i in range(3)]
            sd_ = []
            for i in range(3):
                gd_[i].wait()
                sd_.append(pltpu.async_copy(bufs[i], acc.at[didx_v.at[RPW - 3 + i]],
                                            ssems[i], add=True))
            for i in range(3):
                sd_[i].wait()

        plsc.subcore_barrier()
        pltpu.sync_copy(acc.at[sl], out_h.at[cid, sl])

        @pl.when(sid == NS - 1)
        def _otail():
            pltpu.sync_copy(acc.at[tl], out_h.at[cid, tl])

    return k(h, src2d, dst2d, zeros_nd)


_BM = 1000


def _tc_mm1(x, W1):
    def body(x_ref, w_ref, o_ref):
        o_ref[...] = jnp.dot(x_ref[...], w_ref[...], preferred_element_type=F32,
                             precision=lax.Precision.HIGHEST)

    return pl.pallas_call(
        body,
        grid=(N // _BM,),
        in_specs=[
            pl.BlockSpec((_BM, D_IN), lambda i: (i, 0)),
            pl.BlockSpec((D_IN, H1), lambda i: (0, 0)),
        ],
        out_specs=pl.BlockSpec((_BM, H1), lambda i: (i, 0)),
        out_shape=jax.ShapeDtypeStruct((N, H1), F32),
    )(x, W1)


def _tc_scale1(u, hist):
    def body(u_ref, hs_ref, o_ref):
        deg = hs_ref[0, 0, :, 0] + hs_ref[1, 0, :, 0]
        o_ref[...] = u_ref[...] * lax.rsqrt(jnp.maximum(deg, 1.0))[:, None]

    return pl.pallas_call(
        body,
        grid=(N // _BM,),
        in_specs=[
            pl.BlockSpec((_BM, H1), lambda i: (i, 0)),
            pl.BlockSpec((NC, 1, _BM, DEGW), lambda i: (0, 0, i, 0)),
        ],
        out_specs=pl.BlockSpec((_BM, H1), lambda i: (i, 0)),
        out_shape=jax.ShapeDtypeStruct((N, H1), F32),
    )(u, hist)


def _tc_layer2(agg1, hist, b1, W2):
    def body(p_ref, hin_ref, hout_ref, b_ref, w_ref, o_ref):
        s = p_ref[0] + p_ref[1]
        din = hin_ref[0, 0, :, 0] + hin_ref[1, 0, :, 0]
        dout = hout_ref[0, 0, :, 0] + hout_ref[1, 0, :, 0]
        h = s * lax.rsqrt(jnp.maximum(din, 1.0))[:, None] + b_ref[0][None, :]
        h = jnp.maximum(h, 0.0)
        o_ref[...] = (
            jnp.dot(h, w_ref[...], preferred_element_type=F32,
                    precision=lax.Precision.HIGHEST)
            * lax.rsqrt(jnp.maximum(dout, 1.0))[:, None]
        )

    return pl.pallas_call(
        body,
        grid=(N // _BM,),
        in_specs=[
            pl.BlockSpec((NC, _BM, H1), lambda i: (0, i, 0)),
            pl.BlockSpec((NC, 1, _BM, DEGW), lambda i: (0, 1, i, 0)),
            pl.BlockSpec((NC, 1, _BM, DEGW), lambda i: (0, 0, i, 0)),
            pl.BlockSpec((1, H1), lambda i: (0, 0)),
            pl.BlockSpec((H1, H2), lambda i: (0, 0)),
        ],
        out_specs=pl.BlockSpec((_BM, H2), lambda i: (i, 0)),
        out_shape=jax.ShapeDtypeStruct((N, H2), F32),
    )(agg1, hist, hist, b1, W2)


def _tc_z(agg2, hist, b2):
    def body(p_ref, hin_ref, b_ref, o_ref):
        din = hin_ref[0, 0, :, 0] + hin_ref[1, 0, :, 0]
        o_ref[...] = (
            (p_ref[0] + p_ref[1]) * lax.rsqrt(jnp.maximum(din, 1.0))[:, None]
            + b_ref[0][None, :]
        )

    return pl.pallas_call(
        body,
        grid=(N // _BM,),
        in_specs=[
            pl.BlockSpec((NC, _BM, H2), lambda i: (0, i, 0)),
            pl.BlockSpec((NC, 1, _BM, DEGW), lambda i: (0, 1, i, 0)),
            pl.BlockSpec((1, H2), lambda i: (0, 0)),
        ],
        out_specs=pl.BlockSpec((_BM, H2), lambda i: (i, 0)),
        out_shape=jax.ShapeDtypeStruct((N, H2), F32),
    )(agg2, hist, b2)


_DM = 400


def _tc_decoder(z):
    def body(zi_ref, zall_ref, o_ref):
        o_ref[...] = lax.dot_general(
            zi_ref[...], zall_ref[...], (((1,), (1,)), ((), ())),
            preferred_element_type=F32, precision=lax.Precision.HIGHEST)

    return pl.pallas_call(
        body,
        grid=(N // _DM,),
        in_specs=[
            pl.BlockSpec((_DM, H2), lambda i: (i, 0)),
            pl.BlockSpec((N, H2), lambda i: (0, 0)),
        ],
        out_specs=pl.BlockSpec((_DM, N), lambda i: (i, 0)),
        out_shape=jax.ShapeDtypeStruct((N, N), F32),
    )(z, z)


def kernel(x, edge_index, W1, b1, W2, b2):
    src1d = edge_index[0].reshape(R, EW)
    dst1d = edge_index[1].reshape(R, EW)
    zeros_deg = jnp.zeros((N, DEGW), F32)
    zeros_h1 = jnp.zeros((N, H1), F32)
    zeros_h2 = jnp.zeros((N, H2), F32)
    ones_rows = jnp.zeros((EW, DEGW), F32).at[:, 0].set(1.0)

    u = _tc_mm1(x, W1)
    hist = _sc_degrees(src1d, dst1d, ones_rows, zeros_deg)
    h1p = _tc_scale1(u, hist)
    agg1 = _sc_aggregate(h1p, src1d, dst1d, zeros_h1, H1)
    h2p = _tc_layer2(agg1, hist, b1.reshape(1, H1), W2)
    agg2 = _sc_aggregate(h2p, src1d, dst1d, zeros_h2, H2)
    z = _tc_z(agg2, hist, b2.reshape(1, H2))
    return _tc_decoder(z)


# speedup vs baseline: 3.2577x; 1.0014x over previous
import functools

import jax
import jax.numpy as jnp
from jax import lax
from jax.experimental import pallas as pl
from jax.experimental.pallas import tpu as pltpu
from jax.experimental.pallas import tpu_sc as plsc

N = 10000
E = 160000
D_IN = 128
H1 = 64
H2 = 16

NC, NS = 2, 16
NW = NC * NS
EW = 128
R = E // EW
RPW = R // NW
XT = R - RPW * NW
SLAB = 624
TAIL = N - SLAB * NS
TOFF = SLAB * NS
DEGW = 8

F32 = jnp.float32


def _mesh():
    return plsc.VectorSubcoreMesh(core_axis_name="c", subcore_axis_name="s")


_SC_PARAMS = pltpu.CompilerParams(use_tc_tiling_on_sc=False)

_GUARD = 32768


def _preload_idx(src_h, dst_h, sidx_v, didx_v, wid):
    base = wid * RPW
    pltpu.sync_copy(src_h.at[pl.ds(base, RPW)], sidx_v.at[pl.ds(0, RPW)])
    pltpu.sync_copy(dst_h.at[pl.ds(base, RPW)], didx_v.at[pl.ds(0, RPW)])

    @pl.when(wid < XT)
    def _extra():
        pltpu.sync_copy(src_h.at[pl.ds(RPW * NW + wid, 1)], sidx_v.at[pl.ds(RPW, 1)])
        pltpu.sync_copy(dst_h.at[pl.ds(RPW * NW + wid, 1)], didx_v.at[pl.ds(RPW, 1)])


def _sc_degrees(src2d, dst2d, ones_rows, zeros_deg):
    @functools.partial(
        pl.kernel,
        out_type=jax.ShapeDtypeStruct((NC, 2, N, DEGW), F32),
        mesh=_mesh(),
        compiler_params=_SC_PARAMS,
        scratch_types=[
            pltpu.VMEM((RPW + 1, EW), jnp.int32),
            pltpu.VMEM((RPW + 1, EW), jnp.int32),
            pltpu.VMEM((EW, DEGW), F32),
            pltpu.SemaphoreType.DMA,
            pltpu.SemaphoreType.DMA,
            pltpu.SemaphoreType.DMA,
            pltpu.SemaphoreType.DMA,
            pltpu.VMEM_SHARED((_GUARD,), F32),
            pltpu.VMEM_SHARED((N, DEGW), F32),
            pltpu.VMEM_SHARED((N, DEGW), F32),
        ],
    )
    def k(src_h, dst_h, ones_h, zeros_h, out_h,
          sidx_v, didx_v, ones_v, sem_s, sem_d, sem_s2, sem_d2, _g, acc_s, acc_d):
        cid = lax.axis_index("c")
        sid = lax.axis_index("s")
        wid = sid * NC + cid
        sl = pl.ds(sid * SLAB, SLAB)
        tl = pl.ds(TOFF, TAIL)
        pltpu.sync_copy(zeros_h.at[sl], acc_s.at[sl])
        pltpu.sync_copy(zeros_h.at[sl], acc_d.at[sl])

        @pl.when(sid == NS - 1)
        def _ztail():
            pltpu.sync_copy(zeros_h.at[tl], acc_s.at[tl])
            pltpu.sync_copy(zeros_h.at[tl], acc_d.at[tl])

        pltpu.sync_copy(ones_h, ones_v)
        _preload_idx(src_h, dst_h, sidx_v, didx_v, wid)
        plsc.subcore_barrier()

        npairs = jnp.where(wid < XT, (RPW + 1) // 2, RPW // 2)

        def body(j, carry):
            ra = 2 * j
            rb = 2 * j + 1
            ds_ = (
                pltpu.async_copy(ones_v, acc_s.at[sidx_v.at[ra]], sem_s, add=True),
                pltpu.async_copy(ones_v, acc_d.at[didx_v.at[ra]], sem_d, add=True),
                pltpu.async_copy(ones_v, acc_s.at[sidx_v.at[rb]], sem_s2, add=True),
                pltpu.async_copy(ones_v, acc_d.at[didx_v.at[rb]], sem_d2, add=True),
            )
            for d in ds_:
                d.wait()
            return carry

        lax.fori_loop(0, npairs, body, None)

        @pl.when(wid >= XT)
        def _oddrow():
            d1 = pltpu.async_copy(ones_v, acc_s.at[sidx_v.at[RPW - 1]], sem_s, add=True)
            d2 = pltpu.async_copy(ones_v, acc_d.at[didx_v.at[RPW - 1]], sem_d, add=True)
            d1.wait()
            d2.wait()
        plsc.subcore_barrier()
        pltpu.sync_copy(acc_s.at[sl], out_h.at[cid, 0, sl])
        pltpu.sync_copy(acc_d.at[sl], out_h.at[cid, 1, sl])

        @pl.when(sid == NS - 1)
        def _otail():
            pltpu.sync_copy(acc_s.at[tl], out_h.at[cid, 0, tl])
            pltpu.sync_copy(acc_d.at[tl], out_h.at[cid, 1, tl])

    return k(src2d, dst2d, ones_rows, zeros_deg)


def _sc_aggregate(h, src2d, dst2d, zeros_nd, D):
    @functools.partial(
        pl.kernel,
        out_type=jax.ShapeDtypeStruct((NC, N, D), F32),
        mesh=_mesh(),
        compiler_params=_SC_PARAMS,
        scratch_types=[
            pltpu.VMEM((RPW + 1, EW), jnp.int32),
            pltpu.VMEM((RPW + 1, EW), jnp.int32),
            pltpu.VMEM((EW, D), F32),
            pltpu.VMEM((EW, D), F32),
            pltpu.VMEM((EW, D), F32),
            pltpu.VMEM((EW, D), F32),
            pltpu.SemaphoreType.DMA,
            pltpu.SemaphoreType.DMA,
            pltpu.SemaphoreType.DMA,
            pltpu.SemaphoreType.DMA,
            pltpu.SemaphoreType.DMA,
            pltpu.SemaphoreType.DMA,
            pltpu.SemaphoreType.DMA,
            pltpu.SemaphoreType.DMA,
            pltpu.VMEM_SHARED((_GUARD,), F32),
            pltpu.VMEM_SHARED((N, D), F32),
        ],
    )
    def k(h_h, src_h, dst_h, zeros_h, out_h,
          sidx_v, didx_v, rows_a, rows_b, rows_c, rows_d,
          ga, gb, gc, gd, sa, sb, sc_, sd, _g, acc):
        cid = lax.axis_index("c")
        sid = lax.axis_index("s")
        wid = sid * NC + cid
        sl = pl.ds(sid * SLAB, SLAB)
        tl = pl.ds(TOFF, TAIL)
        pltpu.sync_copy(zeros_h.at[sl], acc.at[sl])

        @pl.when(sid == NS - 1)
        def _ztail():
            pltpu.sync_copy(zeros_h.at[tl], acc.at[tl])

        _preload_idx(src_h, dst_h, sidx_v, didx_v, wid)
        plsc.subcore_barrier()

        nquads = jnp.where(wid < XT, (RPW + 1) // 4, RPW // 4)
        bufs = (rows_a, rows_b, rows_c, rows_d)
        gsems = (ga, gb, gc, gd)
        ssems = (sa, sb, sc_, sd)

        def body(j, carry):
            gd_ = [pltpu.async_copy(h_h.at[sidx_v.at[4 * j + i]], bufs[i], gsems[i])
                   for i in range(4)]
            sd_ = []
            for i in range(4):
                gd_[i].wait()
                sd_.append(pltpu.async_copy(bufs[i], acc.at[didx_v.at[4 * j + i]],
                                            ssems[i], add=True))
            for i in range(4):
                sd_[i].wait()
            return carry

        lax.fori_loop(0, nquads, body, None)

        @pl.when(wid >= XT)
        def _odd():
            gd_ = [pltpu.async_copy(h_h.at[sidx_v.at[RPW - 3 + i]], bufs[i], gsems[i])
                   for i in range(3)]
            sd_ = []
            for i in range(3):
                gd_[i].wait()
                sd_.append(pltpu.async_copy(bufs[i], acc.at[didx_v.at[RPW - 3 + i]],
                                            ssems[i], add=True))
            for i in range(3):
                sd_[i].wait()

        plsc.subcore_barrier()
        pltpu.sync_copy(acc.at[sl], out_h.at[cid, sl])

        @pl.when(sid == NS - 1)
        def _otail():
            pltpu.sync_copy(acc.at[tl], out_h.at[cid, tl])

    return k(h, src2d, dst2d, zeros_nd)


_BM = 1000


def _tc_mm1(x, W1):
    def body(x_ref, w_ref, o_ref):
        o_ref[...] = jnp.dot(x_ref[...], w_ref[...], preferred_element_type=F32,
                             precision=lax.Precision.HIGHEST)

    return pl.pallas_call(
        body,
        grid=(N // _BM,),
        in_specs=[
            pl.BlockSpec((_BM, D_IN), lambda i: (i, 0)),
            pl.BlockSpec((D_IN, H1), lambda i: (0, 0)),
        ],
        out_specs=pl.BlockSpec((_BM, H1), lambda i: (i, 0)),
        out_shape=jax.ShapeDtypeStruct((N, H1), F32),
    )(x, W1)


def _tc_scale1(u, hist):
    def body(u_ref, hs_ref, o_ref):
        deg = hs_ref[0, 0, :, 0] + hs_ref[1, 0, :, 0]
        o_ref[...] = u_ref[...] * lax.rsqrt(jnp.maximum(deg, 1.0))[:, None]

    return pl.pallas_call(
        body,
        grid=(N // _BM,),
        in_specs=[
            pl.BlockSpec((_BM, H1), lambda i: (i, 0)),
            pl.BlockSpec((NC, 1, _BM, DEGW), lambda i: (0, 0, i, 0)),
        ],
        out_specs=pl.BlockSpec((_BM, H1), lambda i: (i, 0)),
        out_shape=jax.ShapeDtypeStruct((N, H1), F32),
    )(u, hist)


def _tc_layer2(agg1, hist, b1, W2):
    def body(p_ref, hin_ref, hout_ref, b_ref, w_ref, o_ref):
        s = p_ref[0] + p_ref[1]
        din = hin_ref[0, 0, :, 0] + hin_ref[1, 0, :, 0]
        dout = hout_ref[0, 0, :, 0] + hout_ref[1, 0, :, 0]
        h = s * lax.rsqrt(jnp.maximum(din, 1.0))[:, None] + b_ref[0][None, :]
        h = jnp.maximum(h, 0.0)
        o_ref[...] = (
            jnp.dot(h, w_ref[...], preferred_element_type=F32,
                    precision=lax.Precision.HIGHEST)
            * lax.rsqrt(jnp.maximum(dout, 1.0))[:, None]
        )

    return pl.pallas_call(
        body,
        grid=(N // _BM,),
        in_specs=[
            pl.BlockSpec((NC, _BM, H1), lambda i: (0, i, 0)),
            pl.BlockSpec((NC, 1, _BM, DEGW), lambda i: (0, 1, i, 0)),
            pl.BlockSpec((NC, 1, _BM, DEGW), lambda i: (0, 0, i, 0)),
            pl.BlockSpec((1, H1), lambda i: (0, 0)),
            pl.BlockSpec((H1, H2), lambda i: (0, 0)),
        ],
        out_specs=pl.BlockSpec((_BM, H2), lambda i: (i, 0)),
        out_shape=jax.ShapeDtypeStruct((N, H2), F32),
    )(agg1, hist, hist, b1, W2)


def _tc_z(agg2, hist, b2):
    def body(p_ref, hin_ref, b_ref, o_ref):
        din = hin_ref[0, 0, :, 0] + hin_ref[1, 0, :, 0]
        o_ref[...] = (
            (p_ref[0] + p_ref[1]) * lax.rsqrt(jnp.maximum(din, 1.0))[:, None]
            + b_ref[0][None, :]
        )

    return pl.pallas_call(
        body,
        grid=(N // _BM,),
        in_specs=[
            pl.BlockSpec((NC, _BM, H2), lambda i: (0, i, 0)),
            pl.BlockSpec((NC, 1, _BM, DEGW), lambda i: (0, 1, i, 0)),
            pl.BlockSpec((1, H2), lambda i: (0, 0)),
        ],
        out_specs=pl.BlockSpec((_BM, H2), lambda i: (i, 0)),
        out_shape=jax.ShapeDtypeStruct((N, H2), F32),
    )(agg2, hist, b2)


_DM = 400


def _tc_decoder(z):
    def body(zi_ref, zall_ref, o_ref):
        o_ref[...] = lax.dot_general(
            zi_ref[...], zall_ref[...], (((1,), (1,)), ((), ())),
            preferred_element_type=F32, precision=lax.Precision.HIGHEST)

    return pl.pallas_call(
        body,
        grid=(N // _DM,),
        in_specs=[
            pl.BlockSpec((_DM, H2), lambda i: (i, 0)),
            pl.BlockSpec((N, H2), lambda i: (0, 0)),
        ],
        out_specs=pl.BlockSpec((_DM, N), lambda i: (i, 0)),
        out_shape=jax.ShapeDtypeStruct((N, N), F32),
    )(z, z)


def kernel(x, edge_index, W1, b1, W2, b2):
    src1d = edge_index[0].reshape(R, EW)
    dst1d = edge_index[1].reshape(R, EW)
    zeros_deg = jnp.zeros((N, DEGW), F32)
    zeros_h1 = jnp.zeros((N, H1), F32)
    zeros_h2 = jnp.zeros((N, H2), F32)
    ones_rows = jnp.zeros((EW, DEGW), F32).at[:, 0].set(1.0)

    u = _tc_mm1(x, W1)
    hist = _sc_degrees(src1d, dst1d, ones_rows, zeros_deg)
    h1p = _tc_scale1(u, hist)
    agg1 = _sc_aggregate(h1p, src1d, dst1d, zeros_h1, H1)
    h2p = _tc_layer2(agg1, hist, b1.reshape(1, H1), W2)
    agg2 = _sc_aggregate(h2p, src1d, dst1d, zeros_h2, H2)
    z = _tc_z(agg2, hist, b2.reshape(1, H2))
    return _tc_decoder(z)
